# baseline (speedup 1.0000x reference)
import functools

import jax
import jax.numpy as jnp
from jax import lax
from jax.experimental import pallas as pl
from jax.experimental.pallas import tpu as pltpu
from jax.experimental.pallas import tpu_sc as plsc

VOCAB = 100000
HIDDEN = 1024
B = 4
S = 8192
N_IMG = 1024

NC = 2
NS = 16
NW = NC * NS
ROWS = B * S
RPW = ROWS // NW
CHUNK = 32
NCHUNK = RPW // CHUNK
DUMMY = ROWS
R_PAD = ROWS + 8
CAP_W = N_IMG // NS
P2_CHUNKS = CAP_W // CHUNK


def _sc_body(ids_hbm, dest_hbm, lcidx_hbm, wte_hbm, lcf_hbm, out_hbm,
             idsv, destv, lcidxv, rows, gsem):
    c = lax.axis_index("c")
    s = lax.axis_index("s")
    wid = s * NC + c
    base = wid * RPW

    pltpu.sync_copy(ids_hbm.at[pl.ds(base, RPW)], idsv)

    @pl.loop(0, NCHUNK)
    def _phase1(ci):
        pltpu.async_copy(
            wte_hbm.at[idsv.at[pl.ds(ci * CHUNK, CHUNK)]], rows, gsem
        ).wait()
        pltpu.sync_copy(rows, out_hbm.at[pl.ds(base + ci * CHUNK, CHUNK)])

    plsc.subcore_barrier()

    pltpu.sync_copy(dest_hbm.at[wid], destv)
    pltpu.sync_copy(lcidx_hbm.at[wid], lcidxv)

    @pl.loop(0, P2_CHUNKS)
    def _phase2(k):
        pltpu.async_copy(lcf_hbm.at[lcidxv.at[k]], rows, gsem).wait()
        pltpu.async_copy(rows, out_hbm.at[destv.at[k]], gsem).wait()


def _mm_body(lc_ref, w_ref, b_ref, o_ref):
    o_ref[...] = (
        jnp.dot(lc_ref[...], w_ref[...], preferred_element_type=jnp.float32)
        + b_ref[...]
    )


def _prep_scatter(pos_batch, pos_seq):
    j = jnp.arange(N_IMG, dtype=jnp.int32)
    winner = jnp.full((B, S), -1, jnp.int32).at[pos_batch, pos_seq].set(j)
    keep = winner[pos_batch, pos_seq] == j
    flat = (pos_batch.astype(jnp.int32) * S + pos_seq.astype(jnp.int32))
    owner_c = (flat // RPW) % NC

    dest_full = jnp.full((NC * N_IMG,), DUMMY, jnp.int32)
    lcidx_full = jnp.zeros((NC * N_IMG,), jnp.int32)
    for core in range(NC):
        m = keep & (owner_c == core)
        rank = jnp.cumsum(m.astype(jnp.int32)) - 1
        slot = jnp.where(m, core * N_IMG + rank, NC * N_IMG)
        dest_full = dest_full.at[slot].set(flat, mode="drop")
        lcidx_full = lcidx_full.at[slot].set(j, mode="drop")

    dest_arr = (
        dest_full.reshape(NC, NS, P2_CHUNKS, CHUNK)
        .transpose(1, 0, 2, 3)
        .reshape(NW, P2_CHUNKS, CHUNK)
    )
    lcidx_arr = (
        lcidx_full.reshape(NC, NS, P2_CHUNKS, CHUNK)
        .transpose(1, 0, 2, 3)
        .reshape(NW, P2_CHUNKS, CHUNK)
    )
    return dest_arr, lcidx_arr


@functools.cache
def _build_sc_kernel():
    mesh = plsc.VectorSubcoreMesh(
        core_axis_name="c", subcore_axis_name="s", num_cores=NC,
        num_subcores=NS,
    )
    return pl.kernel(
        _sc_body,
        out_type=jax.ShapeDtypeStruct((R_PAD, HIDDEN), jnp.float32),
        mesh=mesh,
        scratch_types=[
            pltpu.VMEM((RPW,), jnp.int32),
            pltpu.VMEM((P2_CHUNKS, CHUNK), jnp.int32),
            pltpu.VMEM((P2_CHUNKS, CHUNK), jnp.int32),
            pltpu.VMEM((CHUNK, HIDDEN), jnp.float32),
            pltpu.SemaphoreType.DMA,
        ],
    )


def kernel(input_ids, lc_values, pos_batch, pos_seq, wte, W, b):
    ids = jnp.clip(input_ids.astype(jnp.int32), 0, VOCAB).reshape(-1)
    dest_arr, lcidx_arr = _prep_scatter(pos_batch, pos_seq)

    lc_features = pl.pallas_call(
        _mm_body,
        out_shape=jax.ShapeDtypeStruct((N_IMG, HIDDEN), jnp.float32),
    )(lc_values, W, b.reshape(1, HIDDEN))

    out = _build_sc_kernel()(ids, dest_arr, lcidx_arr, wte, lc_features)
    return out[:ROWS].reshape(B, S, HIDDEN)


# speedup vs baseline: 1.0722x; 1.0722x over previous
import functools

import jax
import jax.numpy as jnp
from jax import lax
from jax.experimental import pallas as pl
from jax.experimental.pallas import tpu as pltpu
from jax.experimental.pallas import tpu_sc as plsc

VOCAB = 100000
HIDDEN = 1024
B = 4
S = 8192
N_IMG = 1024

NC = 2
NS = 16
NW = NC * NS
ROWS = B * S
RPW = ROWS // NW
CHUNK = 32
NCHUNK = RPW // CHUNK
DUMMY = ROWS
R_PAD = ROWS + 8
CAP_W = N_IMG // NS
P2_CHUNKS = CAP_W // CHUNK


def _sc_body(ids_hbm, dest_hbm, lcidx_hbm, wte_hbm, lcf_hbm, out_hbm,
             idsv, destv, lcidxv, rows0, rows1, gsem0, gsem1):
    c = lax.axis_index("c")
    s = lax.axis_index("s")
    wid = s * NC + c
    base = wid * RPW

    pltpu.sync_copy(ids_hbm.at[pl.ds(base, RPW)], idsv)

    bufs = ((rows0, gsem0), (rows1, gsem1))
    for b, (buf, gsem) in enumerate(bufs):
        pltpu.async_copy(wte_hbm.at[idsv.at[pl.ds(b * CHUNK, CHUNK)]],
                         buf, gsem)

    @pl.loop(0, NCHUNK, step=2)
    def _phase1(ci):
        for b, (buf, gsem) in enumerate(bufs):
            cc = ci + b
            pltpu.make_async_copy(
                wte_hbm.at[idsv.at[pl.ds(cc * CHUNK, CHUNK)]], buf, gsem
            ).wait()
            pltpu.sync_copy(buf, out_hbm.at[pl.ds(base + cc * CHUNK, CHUNK)])

            @pl.when(cc + 2 < NCHUNK)
            def _prefetch():
                pltpu.async_copy(
                    wte_hbm.at[idsv.at[pl.ds((cc + 2) * CHUNK, CHUNK)]],
                    buf, gsem)

    plsc.subcore_barrier()

    pltpu.sync_copy(dest_hbm.at[wid], destv)
    pltpu.sync_copy(lcidx_hbm.at[wid], lcidxv)

    @pl.loop(0, P2_CHUNKS)
    def _phase2(k):
        pltpu.async_copy(lcf_hbm.at[lcidxv.at[k]], rows0, gsem0).wait()
        pltpu.async_copy(rows0, out_hbm.at[destv.at[k]], gsem0).wait()


def _mm_body(lc_ref, w_ref, b_ref, o_ref):
    o_ref[...] = (
        jnp.dot(lc_ref[...], w_ref[...], preferred_element_type=jnp.float32)
        + b_ref[...]
    )


def _prep_scatter(pos_batch, pos_seq):
    j = jnp.arange(N_IMG, dtype=jnp.int32)
    winner = jnp.full((B, S), -1, jnp.int32).at[pos_batch, pos_seq].set(j)
    keep = winner[pos_batch, pos_seq] == j
    flat = (pos_batch.astype(jnp.int32) * S + pos_seq.astype(jnp.int32))
    owner_c = (flat // RPW) % NC

    dest_full = jnp.full((NC * N_IMG,), DUMMY, jnp.int32)
    lcidx_full = jnp.zeros((NC * N_IMG,), jnp.int32)
    for core in range(NC):
        m = keep & (owner_c == core)
        rank = jnp.cumsum(m.astype(jnp.int32)) - 1
        slot = jnp.where(m, core * N_IMG + rank, NC * N_IMG)
        dest_full = dest_full.at[slot].set(flat, mode="drop")
        lcidx_full = lcidx_full.at[slot].set(j, mode="drop")

    dest_arr = (
        dest_full.reshape(NC, NS, P2_CHUNKS, CHUNK)
        .transpose(1, 0, 2, 3)
        .reshape(NW, P2_CHUNKS, CHUNK)
    )
    lcidx_arr = (
        lcidx_full.reshape(NC, NS, P2_CHUNKS, CHUNK)
        .transpose(1, 0, 2, 3)
        .reshape(NW, P2_CHUNKS, CHUNK)
    )
    return dest_arr, lcidx_arr


@functools.cache
def _build_sc_kernel():
    mesh = plsc.VectorSubcoreMesh(
        core_axis_name="c", subcore_axis_name="s", num_cores=NC,
        num_subcores=NS,
    )
    return pl.kernel(
        _sc_body,
        out_type=jax.ShapeDtypeStruct((R_PAD, HIDDEN), jnp.float32),
        mesh=mesh,
        scratch_types=[
            pltpu.VMEM((RPW,), jnp.int32),
            pltpu.VMEM((P2_CHUNKS, CHUNK), jnp.int32),
            pltpu.VMEM((P2_CHUNKS, CHUNK), jnp.int32),
            pltpu.VMEM((CHUNK, HIDDEN), jnp.float32),
            pltpu.VMEM((CHUNK, HIDDEN), jnp.float32),
            pltpu.SemaphoreType.DMA,
            pltpu.SemaphoreType.DMA,
        ],
    )


def kernel(input_ids, lc_values, pos_batch, pos_seq, wte, W, b):
    ids = jnp.clip(input_ids.astype(jnp.int32), 0, VOCAB).reshape(-1)
    dest_arr, lcidx_arr = _prep_scatter(pos_batch, pos_seq)

    lc_features = pl.pallas_call(
        _mm_body,
        out_shape=jax.ShapeDtypeStruct((N_IMG, HIDDEN), jnp.float32),
    )(lc_values, W, b.reshape(1, HIDDEN))

    out = _build_sc_kernel()(ids, dest_arr, lcidx_arr, wte, lc_features)
    return out[:ROWS].reshape(B, S, HIDDEN)


# speedup vs baseline: 2.7737x; 2.5869x over previous
import functools

import jax
import jax.numpy as jnp
from jax import lax
from jax.experimental import pallas as pl
from jax.experimental.pallas import tpu as pltpu
from jax.experimental.pallas import tpu_sc as plsc

VOCAB = 100000
HIDDEN = 1024
B = 4
S = 8192
N_IMG = 1024

NC = 2
NS = 16
NW = NC * NS
ROWS = B * S
RPW = ROWS // NW
CHUNK = 32
NCHUNK = RPW // CHUNK
SCAT_W = N_IMG // NW


def _gather_body(ids_hbm, wte_hbm, out_hbm, idsv, rows0, rows1, gsem0, gsem1):
    c = lax.axis_index("c")
    s = lax.axis_index("s")
    wid = s * NC + c
    base = wid * RPW

    pltpu.sync_copy(ids_hbm.at[pl.ds(base, RPW)], idsv)

    bufs = ((rows0, gsem0), (rows1, gsem1))
    for b, (buf, gsem) in enumerate(bufs):
        pltpu.async_copy(wte_hbm.at[idsv.at[pl.ds(b * CHUNK, CHUNK)]],
                         buf, gsem)

    @pl.loop(0, NCHUNK, step=2)
    def _phase1(ci):
        for b, (buf, gsem) in enumerate(bufs):
            cc = ci + b
            pltpu.make_async_copy(
                wte_hbm.at[idsv.at[pl.ds(cc * CHUNK, CHUNK)]], buf, gsem
            ).wait()
            pltpu.sync_copy(buf, out_hbm.at[pl.ds(base + cc * CHUNK, CHUNK)])

            @pl.when(cc + 2 < NCHUNK)
            def _prefetch():
                pltpu.async_copy(
                    wte_hbm.at[idsv.at[pl.ds((cc + 2) * CHUNK, CHUNK)]],
                    buf, gsem)


def _scatter_body(dest_hbm, lcidx_hbm, lcf_hbm, out_ref, destv, lcidxv,
                  rows, gsem):
    c = lax.axis_index("c")
    s = lax.axis_index("s")
    wid = s * NC + c
    pltpu.sync_copy(dest_hbm.at[wid], destv)
    pltpu.sync_copy(lcidx_hbm.at[wid], lcidxv)
    pltpu.async_copy(lcf_hbm.at[lcidxv], rows, gsem).wait()
    pltpu.async_copy(rows, out_ref.at[destv], gsem).wait()


def _mm_body(lc_ref, w_ref, b_ref, o_ref):
    o_ref[...] = (
        jnp.dot(lc_ref[...], w_ref[...], preferred_element_type=jnp.float32)
        + b_ref[...]
    )


def _prep_scatter(pos_batch, pos_seq):
    j = jnp.arange(N_IMG, dtype=jnp.int32)
    winner = jnp.full((B, S), -1, jnp.int32).at[pos_batch, pos_seq].set(j)
    keep = winner[pos_batch, pos_seq] == j
    flat = pos_batch.astype(jnp.int32) * S + pos_seq.astype(jnp.int32)

    rank = jnp.cumsum(keep.astype(jnp.int32)) - 1
    n = rank[-1] + 1
    slot = jnp.where(keep, rank, N_IMG)
    pairs = jnp.zeros((N_IMG, 2), jnp.int32).at[slot].set(
        jnp.stack([flat, j], axis=1), mode="drop")
    pairs = jnp.where(j[:, None] < n, pairs, pairs[0])

    per_w = pairs.reshape(NW, SCAT_W, 2)
    return per_w[:, :, 0], per_w[:, :, 1]


@functools.cache
def _build_kernels():
    mesh = plsc.VectorSubcoreMesh(
        core_axis_name="c", subcore_axis_name="s", num_cores=NC,
        num_subcores=NS,
    )
    gather = pl.kernel(
        _gather_body,
        out_type=jax.ShapeDtypeStruct((ROWS, HIDDEN), jnp.float32),
        mesh=mesh,
        scratch_types=[
            pltpu.VMEM((RPW,), jnp.int32),
            pltpu.VMEM((CHUNK, HIDDEN), jnp.float32),
            pltpu.VMEM((CHUNK, HIDDEN), jnp.float32),
            pltpu.SemaphoreType.DMA,
            pltpu.SemaphoreType.DMA,
        ],
    )
    scatter = pl.kernel(
        _scatter_body,
        out_type=(),
        mesh=mesh,
        scratch_types=[
            pltpu.VMEM((SCAT_W,), jnp.int32),
            pltpu.VMEM((SCAT_W,), jnp.int32),
            pltpu.VMEM((SCAT_W, HIDDEN), jnp.float32),
            pltpu.SemaphoreType.DMA,
        ],
    )
    return gather, scatter


def kernel(input_ids, lc_values, pos_batch, pos_seq, wte, W, b):
    gather, scatter = _build_kernels()
    ids = jnp.clip(input_ids.astype(jnp.int32), 0, VOCAB).reshape(-1)
    dest_arr, lcidx_arr = _prep_scatter(pos_batch, pos_seq)

    lc_features = pl.pallas_call(
        _mm_body,
        out_shape=jax.ShapeDtypeStruct((N_IMG, HIDDEN), jnp.float32),
    )(lc_values, W, b.reshape(1, HIDDEN))

    out = gather(ids, wte)
    out_ref = jax.new_ref(out)
    scatter(dest_arr, lcidx_arr, lc_features, out_ref)
    return out_ref[...].reshape(B, S, HIDDEN)
